# baseline (speedup 1.0000x reference)
import functools

import jax
import jax.numpy as jnp
from jax import lax
from jax.experimental import pallas as pl
from jax.experimental.pallas import tpu as pltpu
from jax.experimental.pallas import tpu_sc as plsc

N, E, D, H, L = 10000, 160000, 256, 512, 3
NC, NS = 2, 16
CH = 128
TCHUNKS = 1280
EPAD = TCHUNKS * CH
CHS = 256
SCHUNKS = EPAD // CHS
GARBAGE = N
ACC_ROWS = N + 16
ZPT = ACC_ROWS // NS
RPT = N // NS
HD = D // 2
BN = 1000
GRID = N // BN


@functools.cache
def _sc_kernels():
    mesh = plsc.VectorSubcoreMesh(
        core_axis_name="c", subcore_axis_name="s", num_cores=NC, num_subcores=NS
    )

    params = pltpu.CompilerParams(use_tc_tiling_on_sc=False)

    cpt = SCHUNKS // NS
    pf = 2
    nbuf = 4

    @functools.partial(
        pl.kernel,
        out_type=(jax.ShapeDtypeStruct((N, HD), jnp.bfloat16),
                  jax.ShapeDtypeStruct((N, HD), jnp.bfloat16)),
        mesh=mesh,
        compiler_params=params,
        scratch_types=[
            pltpu.VMEM_SHARED((ACC_ROWS, HD), jnp.bfloat16),
            pltpu.VMEM((cpt, CHS), jnp.int32),
            pltpu.VMEM((cpt, CHS), jnp.int32),
            [pltpu.VMEM((CHS, HD), jnp.bfloat16)] * nbuf,
            [pltpu.SemaphoreType.DMA] * nbuf,
            [pltpu.SemaphoreType.DMA] * nbuf,
        ],
    )
    def _seg_sum(rp_lo, rp_hi, sidx_hbm, didx_hbm, zeros_hbm,
                 out_lo, out_hi, acc, sidx_buf, didx_buf, rows, gsems, ssems):
        c = lax.axis_index("c")
        s = lax.axis_index("s")

        pltpu.sync_copy(zeros_hbm.at[pl.ds(s * ZPT, ZPT)],
                        acc.at[pl.ds(s * ZPT, ZPT)])
        pltpu.sync_copy(sidx_hbm.at[pl.ds(s * cpt, cpt)], sidx_buf)
        pltpu.sync_copy(didx_hbm.at[pl.ds(s * cpt, cpt)], didx_buf)
        plsc.subcore_barrier()

        def run(rp_hbm, out_hbm):
            def gather(j, b):
                pltpu.async_copy(rp_hbm.at[sidx_buf.at[j]], rows[b], gsems[b])

            def gather_wait(b):
                pltpu.make_async_copy(rp_hbm.at[sidx_buf.at[0]], rows[b],
                                      gsems[b]).wait()

            def scatter(j, b):
                pltpu.async_copy(rows[b], acc.at[didx_buf.at[j]], ssems[b],
                                 add=True)

            def scatter_wait(b):
                pltpu.make_async_copy(rows[b], acc.at[didx_buf.at[0]],
                                      ssems[b]).wait()

            slack = nbuf - pf
            for b in range(pf):
                gather(b, b)

            @pl.loop(0, cpt, step=nbuf)
            def _group(g):
                for b in range(nbuf):
                    j = g + b
                    pb = (b + pf) % nbuf
                    @pl.when(j + pf < cpt)
                    def _():
                        @pl.when(j >= slack)
                        def _():
                            scatter_wait(pb)

                        gather(j + pf, pb)

                    gather_wait(b)
                    scatter(j, b)

            for b in range(nbuf):
                scatter_wait(b)
            plsc.subcore_barrier()
            pltpu.sync_copy(acc.at[pl.ds(s * RPT, RPT)],
                            out_hbm.at[pl.ds(s * RPT, RPT)])

        @pl.when(c == 0)
        def _():
            run(rp_lo, out_lo)

        @pl.when(c == 1)
        def _():
            run(rp_hi, out_hi)

    @functools.partial(
        pl.kernel,
        out_type=(jax.ShapeDtypeStruct((N, 16), jnp.float32),
                  jax.ShapeDtypeStruct((N, 16), jnp.float32)),
        mesh=mesh,
        compiler_params=params,
        scratch_types=[
            pltpu.VMEM_SHARED((ACC_ROWS, 16), jnp.float32),
            pltpu.VMEM((CH,), jnp.int32),
            pltpu.VMEM((CH, 16), jnp.float32),
        ],
    )
    def _deg_count(didx_hbm, zeros_hbm, ones_hbm, out_lo, out_hi,
                   acc, idx_v, ones_v):
        c = lax.axis_index("c")
        s = lax.axis_index("s")
        pltpu.sync_copy(zeros_hbm.at[pl.ds(s * ZPT, ZPT)],
                        acc.at[pl.ds(s * ZPT, ZPT)])
        pltpu.sync_copy(ones_hbm, ones_v)
        plsc.subcore_barrier()
        dpt = TCHUNKS // (NC * NS)

        def body(j, carry):
            t = (c * NS + s) * dpt + j
            pltpu.sync_copy(didx_hbm.at[t], idx_v)
            pltpu.sync_copy(ones_v, acc.at[idx_v], add=True)
            return carry

        lax.fori_loop(0, dpt, body, 0)
        plsc.subcore_barrier()

        @pl.when(c == 0)
        def _():
            pltpu.sync_copy(acc.at[pl.ds(s * RPT, RPT)],
                            out_lo.at[pl.ds(s * RPT, RPT)])

        @pl.when(c == 1)
        def _():
            pltpu.sync_copy(acc.at[pl.ds(s * RPT, RPT)],
                            out_hi.at[pl.ds(s * RPT, RPT)])

    return _seg_sum, _deg_count


_SQRT_HALF = 0.7071067811865476


def _gelu(x):
    return 0.5 * x * (1.0 + lax.erf(x * _SQRT_HALF))


def _ln(x, g, b):
    mu = jnp.mean(x, axis=-1, keepdims=True)
    var = jnp.mean((x - mu) ** 2, axis=-1, keepdims=True)
    return (x - mu) * lax.rsqrt(var + 1e-5) * g + b


def _dotT(a, w):
    return lax.dot_general(a, w, (((1,), (1,)), ((), ())),
                           preferred_element_type=jnp.float32)


def _dinv_from(dlo_ref, dhi_ref):
    deg = dlo_ref[:, 0:1] + dhi_ref[:, 0:1]
    return lax.rsqrt(jnp.maximum(deg, 1.0))


def _agg_from(alo_ref, ahi_ref):
    return jnp.concatenate([alo_ref[...], ahi_ref[...]], axis=-1)


def _store_rp(r, rlo_ref, rhi_ref):
    rb = r.astype(jnp.bfloat16)
    rlo_ref[...] = rb[:, :HD]
    rhi_ref[...] = rb[:, HD:]


def _tc_h_body(x_ref, win_ref, bin_ref, h_ref):
    h_ref[...] = _gelu(_dotT(x_ref[...], win_ref[...]) + bin_ref[...])


def _tc_rp_body(h_ref, dlo_ref, dhi_ref, g_ref, b_ref, rlo_ref, rhi_ref):
    r = _ln(h_ref[...], g_ref[...], b_ref[...]) * _dinv_from(dlo_ref, dhi_ref)
    _store_rp(r, rlo_ref, rhi_ref)


def _tc_mid_body(h_ref, alo_ref, ahi_ref, dlo_ref, dhi_ref,
                 w1b_ref, b1_ref, w2_ref, b2_ref, g_ref, b_ref,
                 hn_ref, rlo_ref, rhi_ref):
    dinv = _dinv_from(dlo_ref, dhi_ref)
    t = _dotT(_agg_from(alo_ref, ahi_ref), w1b_ref[...])
    f = _gelu(t * dinv + b1_ref[...])
    hn = h_ref[...] + _dotT(f, w2_ref[...]) + b2_ref[...]
    hn_ref[...] = hn
    r = _ln(hn, g_ref[...], b_ref[...]) * dinv
    _store_rp(r, rlo_ref, rhi_ref)


def _tc_out_body(h_ref, alo_ref, ahi_ref, dlo_ref, dhi_ref,
                 w1b_ref, b1_ref, w2_ref, b2_ref, g_ref, b_ref,
                 wout_ref, bout_ref, o_ref):
    dinv = _dinv_from(dlo_ref, dhi_ref)
    t = _dotT(_agg_from(alo_ref, ahi_ref), w1b_ref[...])
    f = _gelu(t * dinv + b1_ref[...])
    hn = h_ref[...] + _dotT(f, w2_ref[...]) + b2_ref[...]
    r = _ln(hn, g_ref[...], b_ref[...])
    o_ref[...] = _dotT(r, wout_ref[...]) + bout_ref[...]


def _vec_spec(n):
    return pl.BlockSpec((n,), lambda i: (0,))


_ROW_SPEC = pl.BlockSpec((BN, D), lambda i: (i, 0))
_HALF_SPEC = pl.BlockSpec((BN, HD), lambda i: (i, 0))
_D16_SPEC = pl.BlockSpec((BN, 16), lambda i: (i, 0))
_RP_SHAPE = jax.ShapeDtypeStruct((N, HD), jnp.bfloat16)

_tc_h = pl.pallas_call(
    _tc_h_body,
    grid=(GRID,),
    in_specs=[
        _ROW_SPEC,
        pl.BlockSpec((D, D), lambda i: (0, 0)),
        _vec_spec(D),
    ],
    out_specs=_ROW_SPEC,
    out_shape=jax.ShapeDtypeStruct((N, D), jnp.float32),
)

_tc_rp = pl.pallas_call(
    _tc_rp_body,
    grid=(GRID,),
    in_specs=[
        _ROW_SPEC,
        _D16_SPEC,
        _D16_SPEC,
        _vec_spec(D),
        _vec_spec(D),
    ],
    out_specs=[_HALF_SPEC, _HALF_SPEC],
    out_shape=[_RP_SHAPE, _RP_SHAPE],
)

_tc_mid = pl.pallas_call(
    _tc_mid_body,
    grid=(GRID,),
    in_specs=[
        _ROW_SPEC,
        _HALF_SPEC,
        _HALF_SPEC,
        _D16_SPEC,
        _D16_SPEC,
        pl.BlockSpec((H, D), lambda i: (0, 0)),
        _vec_spec(H),
        pl.BlockSpec((D, H), lambda i: (0, 0)),
        _vec_spec(D),
        _vec_spec(D),
        _vec_spec(D),
    ],
    out_specs=[_ROW_SPEC, _HALF_SPEC, _HALF_SPEC],
    out_shape=[
        jax.ShapeDtypeStruct((N, D), jnp.float32),
        _RP_SHAPE,
        _RP_SHAPE,
    ],
)

_tc_out = pl.pallas_call(
    _tc_out_body,
    grid=(GRID,),
    in_specs=[
        _ROW_SPEC,
        _HALF_SPEC,
        _HALF_SPEC,
        _D16_SPEC,
        _D16_SPEC,
        pl.BlockSpec((H, D), lambda i: (0, 0)),
        _vec_spec(H),
        pl.BlockSpec((D, H), lambda i: (0, 0)),
        _vec_spec(D),
        _vec_spec(D),
        _vec_spec(D),
        pl.BlockSpec((D, D), lambda i: (0, 0)),
        _vec_spec(D),
    ],
    out_specs=_ROW_SPEC,
    out_shape=jax.ShapeDtypeStruct((N, D), jnp.float32),
)


def kernel(x, edge_index, Win, b_in, ln_g, ln_b, W1, b1, W2, b2,
           out_g, out_b, Wout, b_out):
    seg_sum, deg_count = _sc_kernels()
    src = edge_index[0]
    dst = edge_index[1]
    pad = EPAD - E
    sidx = jnp.concatenate(
        [src, jnp.zeros((pad,), jnp.int32)]).reshape(SCHUNKS, CHS)
    didx = jnp.concatenate(
        [dst, jnp.full((pad,), GARBAGE, jnp.int32)]).reshape(SCHUNKS, CHS)
    degidx = jnp.concatenate(
        [src, jnp.full((pad,), GARBAGE, jnp.int32)]).reshape(TCHUNKS, CH)
    zeros_hd = jnp.zeros((ACC_ROWS, HD), jnp.bfloat16)
    zeros_16 = jnp.zeros((ACC_ROWS, 16), jnp.float32)
    ones_16 = jnp.ones((CH, 16), jnp.float32)

    dlo, dhi = deg_count(degidx, zeros_16, ones_16)
    h = _tc_h(x, Win, b_in)
    rlo, rhi = _tc_rp(h, dlo, dhi, ln_g[0], ln_b[0])
    W1b = W1.astype(jnp.bfloat16)
    out = None
    for l in range(L):
        alo, ahi = seg_sum(rlo, rhi, sidx, didx, zeros_hd)
        if l < L - 1:
            h, rlo, rhi = _tc_mid(h, alo, ahi, dlo, dhi, W1b[l], b1[l],
                                  W2[l], b2[l], ln_g[l + 1], ln_b[l + 1])
        else:
            out = _tc_out(h, alo, ahi, dlo, dhi, W1b[l], b1[l], W2[l], b2[l],
                          out_g, out_b, Wout, b_out)
    return out


# speedup vs baseline: 1.0012x; 1.0012x over previous
import functools

import jax
import jax.numpy as jnp
from jax import lax
from jax.experimental import pallas as pl
from jax.experimental.pallas import tpu as pltpu
from jax.experimental.pallas import tpu_sc as plsc

N, E, D, H, L = 10000, 160000, 256, 512, 3
NC, NS = 2, 16
CH = 128
TCHUNKS = 1280
EPAD = TCHUNKS * CH
CHS = 256
SCHUNKS = EPAD // CHS
GARBAGE = N
ACC_ROWS = N + 16
ZPT = ACC_ROWS // NS
RPT = N // NS
HD = D // 2
BN = 2000
GRID = N // BN


@functools.cache
def _sc_kernels():
    mesh = plsc.VectorSubcoreMesh(
        core_axis_name="c", subcore_axis_name="s", num_cores=NC, num_subcores=NS
    )

    params = pltpu.CompilerParams(use_tc_tiling_on_sc=False)

    cpt = SCHUNKS // NS
    pf = 2
    nbuf = 4

    @functools.partial(
        pl.kernel,
        out_type=(jax.ShapeDtypeStruct((N, HD), jnp.bfloat16),
                  jax.ShapeDtypeStruct((N, HD), jnp.bfloat16)),
        mesh=mesh,
        compiler_params=params,
        scratch_types=[
            pltpu.VMEM_SHARED((ACC_ROWS, HD), jnp.bfloat16),
            pltpu.VMEM((cpt, CHS), jnp.int32),
            pltpu.VMEM((cpt, CHS), jnp.int32),
            [pltpu.VMEM((CHS, HD), jnp.bfloat16)] * nbuf,
            [pltpu.SemaphoreType.DMA] * nbuf,
            [pltpu.SemaphoreType.DMA] * nbuf,
        ],
    )
    def _seg_sum(rp_lo, rp_hi, sidx_hbm, didx_hbm, zeros_hbm,
                 out_lo, out_hi, acc, sidx_buf, didx_buf, rows, gsems, ssems):
        c = lax.axis_index("c")
        s = lax.axis_index("s")

        pltpu.sync_copy(zeros_hbm.at[pl.ds(s * ZPT, ZPT)],
                        acc.at[pl.ds(s * ZPT, ZPT)])
        pltpu.sync_copy(sidx_hbm.at[pl.ds(s * cpt, cpt)], sidx_buf)
        pltpu.sync_copy(didx_hbm.at[pl.ds(s * cpt, cpt)], didx_buf)
        plsc.subcore_barrier()

        def run(rp_hbm, out_hbm):
            def gather(j, b):
                pltpu.async_copy(rp_hbm.at[sidx_buf.at[j]], rows[b], gsems[b])

            def gather_wait(b):
                pltpu.make_async_copy(rp_hbm.at[sidx_buf.at[0]], rows[b],
                                      gsems[b]).wait()

            def scatter(j, b):
                pltpu.async_copy(rows[b], acc.at[didx_buf.at[j]], ssems[b],
                                 add=True)

            def scatter_wait(b):
                pltpu.make_async_copy(rows[b], acc.at[didx_buf.at[0]],
                                      ssems[b]).wait()

            slack = nbuf - pf
            for b in range(pf):
                gather(b, b)

            @pl.loop(0, cpt, step=nbuf)
            def _group(g):
                for b in range(nbuf):
                    j = g + b
                    pb = (b + pf) % nbuf
                    @pl.when(j + pf < cpt)
                    def _():
                        @pl.when(j >= slack)
                        def _():
                            scatter_wait(pb)

                        gather(j + pf, pb)

                    gather_wait(b)
                    scatter(j, b)

            for b in range(nbuf):
                scatter_wait(b)
            plsc.subcore_barrier()
            pltpu.sync_copy(acc.at[pl.ds(s * RPT, RPT)],
                            out_hbm.at[pl.ds(s * RPT, RPT)])

        @pl.when(c == 0)
        def _():
            run(rp_lo, out_lo)

        @pl.when(c == 1)
        def _():
            run(rp_hi, out_hi)

    @functools.partial(
        pl.kernel,
        out_type=(jax.ShapeDtypeStruct((N, 16), jnp.float32),
                  jax.ShapeDtypeStruct((N, 16), jnp.float32)),
        mesh=mesh,
        compiler_params=params,
        scratch_types=[
            pltpu.VMEM_SHARED((ACC_ROWS, 16), jnp.float32),
            pltpu.VMEM((CH,), jnp.int32),
            pltpu.VMEM((CH, 16), jnp.float32),
        ],
    )
    def _deg_count(didx_hbm, zeros_hbm, ones_hbm, out_lo, out_hi,
                   acc, idx_v, ones_v):
        c = lax.axis_index("c")
        s = lax.axis_index("s")
        pltpu.sync_copy(zeros_hbm.at[pl.ds(s * ZPT, ZPT)],
                        acc.at[pl.ds(s * ZPT, ZPT)])
        pltpu.sync_copy(ones_hbm, ones_v)
        plsc.subcore_barrier()
        dpt = TCHUNKS // (NC * NS)

        def body(j, carry):
            t = (c * NS + s) * dpt + j
            pltpu.sync_copy(didx_hbm.at[t], idx_v)
            pltpu.sync_copy(ones_v, acc.at[idx_v], add=True)
            return carry

        lax.fori_loop(0, dpt, body, 0)
        plsc.subcore_barrier()

        @pl.when(c == 0)
        def _():
            pltpu.sync_copy(acc.at[pl.ds(s * RPT, RPT)],
                            out_lo.at[pl.ds(s * RPT, RPT)])

        @pl.when(c == 1)
        def _():
            pltpu.sync_copy(acc.at[pl.ds(s * RPT, RPT)],
                            out_hi.at[pl.ds(s * RPT, RPT)])

    return _seg_sum, _deg_count


_SQRT_HALF = 0.7071067811865476


def _gelu(x):
    return 0.5 * x * (1.0 + lax.erf(x * _SQRT_HALF))


def _ln(x, g, b):
    mu = jnp.mean(x, axis=-1, keepdims=True)
    var = jnp.mean((x - mu) ** 2, axis=-1, keepdims=True)
    return (x - mu) * lax.rsqrt(var + 1e-5) * g + b


def _dotT(a, w):
    return lax.dot_general(a, w, (((1,), (1,)), ((), ())),
                           preferred_element_type=jnp.float32)


def _dinv_from(dlo_ref, dhi_ref):
    deg = dlo_ref[:, 0:1] + dhi_ref[:, 0:1]
    return lax.rsqrt(jnp.maximum(deg, 1.0))


def _agg_from(alo_ref, ahi_ref):
    return jnp.concatenate([alo_ref[...], ahi_ref[...]], axis=-1)


def _store_rp(r, rlo_ref, rhi_ref):
    rb = r.astype(jnp.bfloat16)
    rlo_ref[...] = rb[:, :HD]
    rhi_ref[...] = rb[:, HD:]


def _tc_h_body(x_ref, win_ref, bin_ref, h_ref):
    h_ref[...] = _gelu(_dotT(x_ref[...], win_ref[...]) + bin_ref[...])


def _tc_rp_body(h_ref, dlo_ref, dhi_ref, g_ref, b_ref, rlo_ref, rhi_ref):
    r = _ln(h_ref[...], g_ref[...], b_ref[...]) * _dinv_from(dlo_ref, dhi_ref)
    _store_rp(r, rlo_ref, rhi_ref)


def _tc_mid_body(h_ref, alo_ref, ahi_ref, dlo_ref, dhi_ref,
                 w1b_ref, b1_ref, w2_ref, b2_ref, g_ref, b_ref,
                 hn_ref, rlo_ref, rhi_ref):
    dinv = _dinv_from(dlo_ref, dhi_ref)
    t = _dotT(_agg_from(alo_ref, ahi_ref), w1b_ref[...])
    f = _gelu(t * dinv + b1_ref[...]).astype(jnp.bfloat16)
    hn = h_ref[...] + _dotT(f, w2_ref[...]) + b2_ref[...]
    hn_ref[...] = hn
    r = _ln(hn, g_ref[...], b_ref[...]) * dinv
    _store_rp(r, rlo_ref, rhi_ref)


def _tc_out_body(h_ref, alo_ref, ahi_ref, dlo_ref, dhi_ref,
                 w1b_ref, b1_ref, w2_ref, b2_ref, g_ref, b_ref,
                 wout_ref, bout_ref, o_ref):
    dinv = _dinv_from(dlo_ref, dhi_ref)
    t = _dotT(_agg_from(alo_ref, ahi_ref), w1b_ref[...])
    f = _gelu(t * dinv + b1_ref[...]).astype(jnp.bfloat16)
    hn = h_ref[...] + _dotT(f, w2_ref[...]) + b2_ref[...]
    r = _ln(hn, g_ref[...], b_ref[...])
    o_ref[...] = _dotT(r, wout_ref[...]) + bout_ref[...]


def _vec_spec(n):
    return pl.BlockSpec((n,), lambda i: (0,))


_ROW_SPEC = pl.BlockSpec((BN, D), lambda i: (i, 0))
_HALF_SPEC = pl.BlockSpec((BN, HD), lambda i: (i, 0))
_D16_SPEC = pl.BlockSpec((BN, 16), lambda i: (i, 0))
_RP_SHAPE = jax.ShapeDtypeStruct((N, HD), jnp.bfloat16)

_tc_h = pl.pallas_call(
    _tc_h_body,
    grid=(GRID,),
    in_specs=[
        _ROW_SPEC,
        pl.BlockSpec((D, D), lambda i: (0, 0)),
        _vec_spec(D),
    ],
    out_specs=_ROW_SPEC,
    out_shape=jax.ShapeDtypeStruct((N, D), jnp.float32),
)

_tc_rp = pl.pallas_call(
    _tc_rp_body,
    grid=(GRID,),
    in_specs=[
        _ROW_SPEC,
        _D16_SPEC,
        _D16_SPEC,
        _vec_spec(D),
        _vec_spec(D),
    ],
    out_specs=[_HALF_SPEC, _HALF_SPEC],
    out_shape=[_RP_SHAPE, _RP_SHAPE],
)

_tc_mid = pl.pallas_call(
    _tc_mid_body,
    grid=(GRID,),
    in_specs=[
        _ROW_SPEC,
        _HALF_SPEC,
        _HALF_SPEC,
        _D16_SPEC,
        _D16_SPEC,
        pl.BlockSpec((H, D), lambda i: (0, 0)),
        _vec_spec(H),
        pl.BlockSpec((D, H), lambda i: (0, 0)),
        _vec_spec(D),
        _vec_spec(D),
        _vec_spec(D),
    ],
    out_specs=[_ROW_SPEC, _HALF_SPEC, _HALF_SPEC],
    out_shape=[
        jax.ShapeDtypeStruct((N, D), jnp.float32),
        _RP_SHAPE,
        _RP_SHAPE,
    ],
)

_tc_out = pl.pallas_call(
    _tc_out_body,
    grid=(GRID,),
    in_specs=[
        _ROW_SPEC,
        _HALF_SPEC,
        _HALF_SPEC,
        _D16_SPEC,
        _D16_SPEC,
        pl.BlockSpec((H, D), lambda i: (0, 0)),
        _vec_spec(H),
        pl.BlockSpec((D, H), lambda i: (0, 0)),
        _vec_spec(D),
        _vec_spec(D),
        _vec_spec(D),
        pl.BlockSpec((D, D), lambda i: (0, 0)),
        _vec_spec(D),
    ],
    out_specs=_ROW_SPEC,
    out_shape=jax.ShapeDtypeStruct((N, D), jnp.float32),
)


def kernel(x, edge_index, Win, b_in, ln_g, ln_b, W1, b1, W2, b2,
           out_g, out_b, Wout, b_out):
    seg_sum, deg_count = _sc_kernels()
    src = edge_index[0]
    dst = edge_index[1]
    pad = EPAD - E
    sidx = jnp.concatenate(
        [src, jnp.zeros((pad,), jnp.int32)]).reshape(SCHUNKS, CHS)
    didx = jnp.concatenate(
        [dst, jnp.full((pad,), GARBAGE, jnp.int32)]).reshape(SCHUNKS, CHS)
    degidx = jnp.concatenate(
        [src, jnp.full((pad,), GARBAGE, jnp.int32)]).reshape(TCHUNKS, CH)
    zeros_hd = jnp.zeros((ACC_ROWS, HD), jnp.bfloat16)
    zeros_16 = jnp.zeros((ACC_ROWS, 16), jnp.float32)
    ones_16 = jnp.ones((CH, 16), jnp.float32)

    dlo, dhi = deg_count(degidx, zeros_16, ones_16)
    h = _tc_h(x, Win, b_in)
    rlo, rhi = _tc_rp(h, dlo, dhi, ln_g[0], ln_b[0])
    W1b = W1.astype(jnp.bfloat16)
    W2b = W2.astype(jnp.bfloat16)
    out = None
    for l in range(L):
        alo, ahi = seg_sum(rlo, rhi, sidx, didx, zeros_hd)
        if l < L - 1:
            h, rlo, rhi = _tc_mid(h, alo, ahi, dlo, dhi, W1b[l], b1[l],
                                  W2b[l], b2[l], ln_g[l + 1], ln_b[l + 1])
        else:
            out = _tc_out(h, alo, ahi, dlo, dhi, W1b[l], b1[l], W2b[l], b2[l],
                          out_g, out_b, Wout, b_out)
    return out


# speedup vs baseline: 1.0066x; 1.0054x over previous
import functools

import jax
import jax.numpy as jnp
from jax import lax
from jax.experimental import pallas as pl
from jax.experimental.pallas import tpu as pltpu
from jax.experimental.pallas import tpu_sc as plsc

N, E, D, H, L = 10000, 160000, 256, 512, 3
NC, NS = 2, 16
CH = 128
TCHUNKS = 1280
EPAD = TCHUNKS * CH
CHS = 256
SCHUNKS = EPAD // CHS
GARBAGE = N
ACC_ROWS = N + 16
ZPT = ACC_ROWS // NS
RPT = N // NS
HD = D // 2
BN = 1000
GRID = N // BN


@functools.cache
def _sc_kernels():
    mesh = plsc.VectorSubcoreMesh(
        core_axis_name="c", subcore_axis_name="s", num_cores=NC, num_subcores=NS
    )

    params = pltpu.CompilerParams(use_tc_tiling_on_sc=False)

    cpt = SCHUNKS // NS
    pf = 2
    nbuf = 4

    @functools.partial(
        pl.kernel,
        out_type=(jax.ShapeDtypeStruct((N, HD), jnp.bfloat16),
                  jax.ShapeDtypeStruct((N, HD), jnp.bfloat16)),
        mesh=mesh,
        compiler_params=params,
        scratch_types=[
            pltpu.VMEM_SHARED((ACC_ROWS, HD), jnp.bfloat16),
            pltpu.VMEM((cpt, CHS), jnp.int32),
            pltpu.VMEM((cpt, CHS), jnp.int32),
            [pltpu.VMEM((CHS, HD), jnp.bfloat16)] * nbuf,
            [pltpu.SemaphoreType.DMA] * nbuf,
            [pltpu.SemaphoreType.DMA] * nbuf,
        ],
    )
    def _seg_sum(rp_lo, rp_hi, sidx_hbm, didx_hbm, zeros_hbm,
                 out_lo, out_hi, acc, sidx_buf, didx_buf, rows, gsems, ssems):
        c = lax.axis_index("c")
        s = lax.axis_index("s")

        pltpu.sync_copy(zeros_hbm.at[pl.ds(s * ZPT, ZPT)],
                        acc.at[pl.ds(s * ZPT, ZPT)])
        pltpu.sync_copy(sidx_hbm.at[pl.ds(s * cpt, cpt)], sidx_buf)
        pltpu.sync_copy(didx_hbm.at[pl.ds(s * cpt, cpt)], didx_buf)
        plsc.subcore_barrier()

        def run(rp_hbm, out_hbm):
            def gather(j, b):
                pltpu.async_copy(rp_hbm.at[sidx_buf.at[j]], rows[b], gsems[b])

            def gather_wait(b):
                pltpu.make_async_copy(rp_hbm.at[sidx_buf.at[0]], rows[b],
                                      gsems[b]).wait()

            def scatter(j, b):
                pltpu.async_copy(rows[b], acc.at[didx_buf.at[j]], ssems[b],
                                 add=True)

            def scatter_wait(b):
                pltpu.make_async_copy(rows[b], acc.at[didx_buf.at[0]],
                                      ssems[b]).wait()

            slack = nbuf - pf
            for b in range(pf):
                gather(b, b)

            @pl.loop(0, cpt, step=nbuf)
            def _group(g):
                for b in range(nbuf):
                    j = g + b
                    pb = (b + pf) % nbuf
                    @pl.when(j + pf < cpt)
                    def _():
                        @pl.when(j >= slack)
                        def _():
                            scatter_wait(pb)

                        gather(j + pf, pb)

                    gather_wait(b)
                    scatter(j, b)

            for b in range(nbuf):
                scatter_wait(b)
            plsc.subcore_barrier()
            pltpu.sync_copy(acc.at[pl.ds(s * RPT, RPT)],
                            out_hbm.at[pl.ds(s * RPT, RPT)])

        @pl.when(c == 0)
        def _():
            run(rp_lo, out_lo)

        @pl.when(c == 1)
        def _():
            run(rp_hi, out_hi)

    @functools.partial(
        pl.kernel,
        out_type=(jax.ShapeDtypeStruct((N, 16), jnp.float32),
                  jax.ShapeDtypeStruct((N, 16), jnp.float32)),
        mesh=mesh,
        compiler_params=params,
        scratch_types=[
            pltpu.VMEM_SHARED((ACC_ROWS, 16), jnp.float32),
            pltpu.VMEM((CH,), jnp.int32),
            pltpu.VMEM((CH, 16), jnp.float32),
        ],
    )
    def _deg_count(didx_hbm, zeros_hbm, ones_hbm, out_lo, out_hi,
                   acc, idx_v, ones_v):
        c = lax.axis_index("c")
        s = lax.axis_index("s")
        pltpu.sync_copy(zeros_hbm.at[pl.ds(s * ZPT, ZPT)],
                        acc.at[pl.ds(s * ZPT, ZPT)])
        pltpu.sync_copy(ones_hbm, ones_v)
        plsc.subcore_barrier()
        dpt = TCHUNKS // (NC * NS)

        def body(j, carry):
            t = (c * NS + s) * dpt + j
            pltpu.sync_copy(didx_hbm.at[t], idx_v)
            pltpu.sync_copy(ones_v, acc.at[idx_v], add=True)
            return carry

        lax.fori_loop(0, dpt, body, 0)
        plsc.subcore_barrier()

        @pl.when(c == 0)
        def _():
            pltpu.sync_copy(acc.at[pl.ds(s * RPT, RPT)],
                            out_lo.at[pl.ds(s * RPT, RPT)])

        @pl.when(c == 1)
        def _():
            pltpu.sync_copy(acc.at[pl.ds(s * RPT, RPT)],
                            out_hi.at[pl.ds(s * RPT, RPT)])

    return _seg_sum, _deg_count


_SQRT_HALF = 0.7071067811865476


def _gelu(x):
    return 0.5 * x * (1.0 + lax.erf(x * _SQRT_HALF))


def _ln(x, g, b):
    mu = jnp.mean(x, axis=-1, keepdims=True)
    var = jnp.mean((x - mu) ** 2, axis=-1, keepdims=True)
    return (x - mu) * lax.rsqrt(var + 1e-5) * g + b


def _dotT(a, w):
    return lax.dot_general(a, w, (((1,), (1,)), ((), ())),
                           preferred_element_type=jnp.float32)


def _dinv_from(dlo_ref, dhi_ref):
    deg = dlo_ref[:, 0:1] + dhi_ref[:, 0:1]
    return lax.rsqrt(jnp.maximum(deg, 1.0))


def _agg_from(alo_ref, ahi_ref):
    return jnp.concatenate([alo_ref[...], ahi_ref[...]], axis=-1)


def _store_rp(r, rlo_ref, rhi_ref):
    rb = r.astype(jnp.bfloat16)
    rlo_ref[...] = rb[:, :HD]
    rhi_ref[...] = rb[:, HD:]


def _tc_h_body(x_ref, win_ref, bin_ref, h_ref):
    h_ref[...] = _gelu(_dotT(x_ref[...], win_ref[...]) + bin_ref[...])


def _tc_rp_body(h_ref, dlo_ref, dhi_ref, g_ref, b_ref, rlo_ref, rhi_ref):
    r = _ln(h_ref[...], g_ref[...], b_ref[...]) * _dinv_from(dlo_ref, dhi_ref)
    _store_rp(r, rlo_ref, rhi_ref)


def _tc_mid_body(h_ref, alo_ref, ahi_ref, dlo_ref, dhi_ref,
                 w1b_ref, b1_ref, w2_ref, b2_ref, g_ref, b_ref,
                 hn_ref, rlo_ref, rhi_ref):
    dinv = _dinv_from(dlo_ref, dhi_ref)
    t = _dotT(_agg_from(alo_ref, ahi_ref), w1b_ref[...])
    f = _gelu(t * dinv + b1_ref[...]).astype(jnp.bfloat16)
    hn = h_ref[...] + _dotT(f, w2_ref[...]) + b2_ref[...]
    hn_ref[...] = hn
    r = _ln(hn, g_ref[...], b_ref[...]) * dinv
    _store_rp(r, rlo_ref, rhi_ref)


def _tc_out_body(h_ref, alo_ref, ahi_ref, dlo_ref, dhi_ref,
                 w1b_ref, b1_ref, w2_ref, b2_ref, g_ref, b_ref,
                 wout_ref, bout_ref, o_ref):
    dinv = _dinv_from(dlo_ref, dhi_ref)
    t = _dotT(_agg_from(alo_ref, ahi_ref), w1b_ref[...])
    f = _gelu(t * dinv + b1_ref[...]).astype(jnp.bfloat16)
    hn = h_ref[...] + _dotT(f, w2_ref[...]) + b2_ref[...]
    r = _ln(hn, g_ref[...], b_ref[...])
    o_ref[...] = _dotT(r, wout_ref[...]) + bout_ref[...]


def _vec_spec(n):
    return pl.BlockSpec((n,), lambda i: (0,))


_ROW_SPEC = pl.BlockSpec((BN, D), lambda i: (i, 0))
_HALF_SPEC = pl.BlockSpec((BN, HD), lambda i: (i, 0))
_D16_SPEC = pl.BlockSpec((BN, 16), lambda i: (i, 0))
_RP_SHAPE = jax.ShapeDtypeStruct((N, HD), jnp.bfloat16)

_tc_h = pl.pallas_call(
    _tc_h_body,
    grid=(GRID,),
    in_specs=[
        _ROW_SPEC,
        pl.BlockSpec((D, D), lambda i: (0, 0)),
        _vec_spec(D),
    ],
    out_specs=_ROW_SPEC,
    out_shape=jax.ShapeDtypeStruct((N, D), jnp.float32),
)

_tc_rp = pl.pallas_call(
    _tc_rp_body,
    grid=(GRID,),
    in_specs=[
        _ROW_SPEC,
        _D16_SPEC,
        _D16_SPEC,
        _vec_spec(D),
        _vec_spec(D),
    ],
    out_specs=[_HALF_SPEC, _HALF_SPEC],
    out_shape=[_RP_SHAPE, _RP_SHAPE],
)

_tc_mid = pl.pallas_call(
    _tc_mid_body,
    grid=(GRID,),
    in_specs=[
        _ROW_SPEC,
        _HALF_SPEC,
        _HALF_SPEC,
        _D16_SPEC,
        _D16_SPEC,
        pl.BlockSpec((H, D), lambda i: (0, 0)),
        _vec_spec(H),
        pl.BlockSpec((D, H), lambda i: (0, 0)),
        _vec_spec(D),
        _vec_spec(D),
        _vec_spec(D),
    ],
    out_specs=[_ROW_SPEC, _HALF_SPEC, _HALF_SPEC],
    out_shape=[
        jax.ShapeDtypeStruct((N, D), jnp.float32),
        _RP_SHAPE,
        _RP_SHAPE,
    ],
)

_tc_out = pl.pallas_call(
    _tc_out_body,
    grid=(GRID,),
    in_specs=[
        _ROW_SPEC,
        _HALF_SPEC,
        _HALF_SPEC,
        _D16_SPEC,
        _D16_SPEC,
        pl.BlockSpec((H, D), lambda i: (0, 0)),
        _vec_spec(H),
        pl.BlockSpec((D, H), lambda i: (0, 0)),
        _vec_spec(D),
        _vec_spec(D),
        _vec_spec(D),
        pl.BlockSpec((D, D), lambda i: (0, 0)),
        _vec_spec(D),
    ],
    out_specs=_ROW_SPEC,
    out_shape=jax.ShapeDtypeStruct((N, D), jnp.float32),
)


def kernel(x, edge_index, Win, b_in, ln_g, ln_b, W1, b1, W2, b2,
           out_g, out_b, Wout, b_out):
    seg_sum, deg_count = _sc_kernels()
    src = edge_index[0]
    dst = edge_index[1]
    pad = EPAD - E
    sidx = jnp.concatenate(
        [src, jnp.zeros((pad,), jnp.int32)]).reshape(SCHUNKS, CHS)
    didx = jnp.concatenate(
        [dst, jnp.full((pad,), GARBAGE, jnp.int32)]).reshape(SCHUNKS, CHS)
    degidx = jnp.concatenate(
        [src, jnp.full((pad,), GARBAGE, jnp.int32)]).reshape(TCHUNKS, CH)
    zeros_hd = jnp.zeros((ACC_ROWS, HD), jnp.bfloat16)
    zeros_16 = jnp.zeros((ACC_ROWS, 16), jnp.float32)
    ones_16 = jnp.ones((CH, 16), jnp.float32)

    dlo, dhi = deg_count(degidx, zeros_16, ones_16)
    h = _tc_h(x, Win, b_in)
    rlo, rhi = _tc_rp(h, dlo, dhi, ln_g[0], ln_b[0])
    W1b = W1.astype(jnp.bfloat16)
    W2b = W2.astype(jnp.bfloat16)
    out = None
    for l in range(L):
        alo, ahi = seg_sum(rlo, rhi, sidx, didx, zeros_hd)
        if l < L - 1:
            h, rlo, rhi = _tc_mid(h, alo, ahi, dlo, dhi, W1b[l], b1[l],
                                  W2b[l], b2[l], ln_g[l + 1], ln_b[l + 1])
        else:
            out = _tc_out(h, alo, ahi, dlo, dhi, W1b[l], b1[l], W2b[l], b2[l],
                          out_g, out_b, Wout, b_out)
    return out


# speedup vs baseline: 1.0254x; 1.0187x over previous
import functools

import jax
import jax.numpy as jnp
from jax import lax
from jax.experimental import pallas as pl
from jax.experimental.pallas import tpu as pltpu
from jax.experimental.pallas import tpu_sc as plsc

N, E, D, H, L = 10000, 160000, 256, 512, 3
NC, NS = 2, 16
CH = 128
TCHUNKS = 1280
EPAD = TCHUNKS * CH
CHS = 256
SCHUNKS = EPAD // CHS
GARBAGE = N
ACC_ROWS = N + 16
ZPT = ACC_ROWS // NS
RPT = N // NS
HD = D // 2
BN = 1000
GRID = N // BN


@functools.cache
def _sc_kernels():
    mesh = plsc.VectorSubcoreMesh(
        core_axis_name="c", subcore_axis_name="s", num_cores=NC, num_subcores=NS
    )

    params = pltpu.CompilerParams(use_tc_tiling_on_sc=False)

    cpt = SCHUNKS // NS
    pf = 2
    nbuf = 4

    @functools.partial(
        pl.kernel,
        out_type=(jax.ShapeDtypeStruct((N, HD), jnp.bfloat16),
                  jax.ShapeDtypeStruct((N, HD), jnp.bfloat16)),
        mesh=mesh,
        compiler_params=params,
        scratch_types=[
            pltpu.VMEM_SHARED((ACC_ROWS, HD), jnp.bfloat16),
            pltpu.VMEM((cpt, CHS), jnp.int32),
            pltpu.VMEM((cpt, CHS), jnp.int32),
            [pltpu.VMEM((CHS, HD), jnp.bfloat16)] * nbuf,
            [pltpu.SemaphoreType.DMA] * nbuf,
            [pltpu.SemaphoreType.DMA] * nbuf,
        ],
    )
    def _seg_sum(rp_lo, rp_hi, sidx_hbm, didx_hbm, zeros_hbm,
                 out_lo, out_hi, acc, sidx_buf, didx_buf, rows, gsems, ssems):
        c = lax.axis_index("c")
        s = lax.axis_index("s")

        pltpu.sync_copy(zeros_hbm.at[pl.ds(s * ZPT, ZPT)],
                        acc.at[pl.ds(s * ZPT, ZPT)])
        pltpu.sync_copy(sidx_hbm.at[pl.ds(s * cpt, cpt)], sidx_buf)
        pltpu.sync_copy(didx_hbm.at[pl.ds(s * cpt, cpt)], didx_buf)
        plsc.subcore_barrier()

        def run(rp_hbm, out_hbm):
            def gather(j, b):
                pltpu.async_copy(rp_hbm.at[sidx_buf.at[j]], rows[b], gsems[b])

            def gather_wait(b):
                pltpu.make_async_copy(rp_hbm.at[sidx_buf.at[0]], rows[b],
                                      gsems[b]).wait()

            def scatter(j, b):
                pltpu.async_copy(rows[b], acc.at[didx_buf.at[j]], ssems[b],
                                 add=True)

            def scatter_wait(b):
                pltpu.make_async_copy(rows[b], acc.at[didx_buf.at[0]],
                                      ssems[b]).wait()

            slack = nbuf - pf
            for b in range(pf):
                gather(b, b)

            @pl.loop(0, cpt, step=nbuf)
            def _group(g):
                for b in range(nbuf):
                    j = g + b
                    pb = (b + pf) % nbuf
                    @pl.when(j + pf < cpt)
                    def _():
                        @pl.when(j >= slack)
                        def _():
                            scatter_wait(pb)

                        gather(j + pf, pb)

                    gather_wait(b)
                    scatter(j, b)

            for b in range(nbuf):
                scatter_wait(b)
            plsc.subcore_barrier()
            pltpu.sync_copy(acc.at[pl.ds(s * RPT, RPT)],
                            out_hbm.at[pl.ds(s * RPT, RPT)])

        @pl.when(c == 0)
        def _():
            run(rp_lo, out_lo)

        @pl.when(c == 1)
        def _():
            run(rp_hi, out_hi)

    @functools.partial(
        pl.kernel,
        out_type=(jax.ShapeDtypeStruct((N, 16), jnp.float32),
                  jax.ShapeDtypeStruct((N, 16), jnp.float32)),
        mesh=mesh,
        compiler_params=params,
        scratch_types=[
            pltpu.VMEM_SHARED((ACC_ROWS, 16), jnp.float32),
            pltpu.VMEM((CH,), jnp.int32),
            pltpu.VMEM((CH, 16), jnp.float32),
        ],
    )
    def _deg_count(didx_hbm, zeros_hbm, ones_hbm, out_lo, out_hi,
                   acc, idx_v, ones_v):
        c = lax.axis_index("c")
        s = lax.axis_index("s")
        pltpu.sync_copy(zeros_hbm.at[pl.ds(s * ZPT, ZPT)],
                        acc.at[pl.ds(s * ZPT, ZPT)])
        pltpu.sync_copy(ones_hbm, ones_v)
        plsc.subcore_barrier()
        dpt = TCHUNKS // (NC * NS)

        def body(j, carry):
            t = (c * NS + s) * dpt + j
            pltpu.sync_copy(didx_hbm.at[t], idx_v)
            pltpu.sync_copy(ones_v, acc.at[idx_v], add=True)
            return carry

        lax.fori_loop(0, dpt, body, 0)
        plsc.subcore_barrier()

        @pl.when(c == 0)
        def _():
            pltpu.sync_copy(acc.at[pl.ds(s * RPT, RPT)],
                            out_lo.at[pl.ds(s * RPT, RPT)])

        @pl.when(c == 1)
        def _():
            pltpu.sync_copy(acc.at[pl.ds(s * RPT, RPT)],
                            out_hi.at[pl.ds(s * RPT, RPT)])

    return _seg_sum, _deg_count


_SQRT_HALF = 0.7071067811865476


def _gelu(x):
    return 0.5 * x * (1.0 + lax.erf(x * _SQRT_HALF))


def _ln(x, g, b):
    mu = jnp.mean(x, axis=-1, keepdims=True)
    var = jnp.mean((x - mu) ** 2, axis=-1, keepdims=True)
    return (x - mu) * lax.rsqrt(var + 1e-5) * g + b


def _dotT(a, w):
    return lax.dot_general(a, w, (((1,), (1,)), ((), ())),
                           preferred_element_type=jnp.float32)


def _dinv_from(dlo_ref, dhi_ref):
    deg = dlo_ref[:, 0:1] + dhi_ref[:, 0:1]
    return lax.rsqrt(jnp.maximum(deg, 1.0))


def _agg_from(alo_ref, ahi_ref):
    return jnp.concatenate([alo_ref[...], ahi_ref[...]], axis=-1)


def _store_rp(r, rlo_ref, rhi_ref):
    rb = r.astype(jnp.bfloat16)
    rlo_ref[...] = rb[:, :HD]
    rhi_ref[...] = rb[:, HD:]


def _tc_h_body(x_ref, win_ref, bin_ref, h_ref):
    h_ref[...] = _gelu(_dotT(x_ref[...], win_ref[...]) + bin_ref[...])


def _tc_rp_body(h_ref, dlo_ref, dhi_ref, g_ref, b_ref, rlo_ref, rhi_ref):
    r = _ln(h_ref[...], g_ref[...], b_ref[...]) * _dinv_from(dlo_ref, dhi_ref)
    _store_rp(r, rlo_ref, rhi_ref)


def _tc_mid_body(h_ref, alo_ref, ahi_ref, dlo_ref, dhi_ref,
                 w1b_ref, b1_ref, w2_ref, b2_ref, g_ref, b_ref,
                 hn_ref, rlo_ref, rhi_ref):
    dinv = _dinv_from(dlo_ref, dhi_ref)
    t = _dotT(_agg_from(alo_ref, ahi_ref), w1b_ref[...])
    f = _gelu(t * dinv + b1_ref[...])
    hn = h_ref[...] + _dotT(f, w2_ref[...]) + b2_ref[...]
    hn_ref[...] = hn
    r = _ln(hn, g_ref[...], b_ref[...]) * dinv
    _store_rp(r, rlo_ref, rhi_ref)


def _tc_out_body(h_ref, alo_ref, ahi_ref, dlo_ref, dhi_ref,
                 w1b_ref, b1_ref, w2_ref, b2_ref, g_ref, b_ref,
                 wout_ref, bout_ref, o_ref):
    dinv = _dinv_from(dlo_ref, dhi_ref)
    t = _dotT(_agg_from(alo_ref, ahi_ref), w1b_ref[...])
    f = _gelu(t * dinv + b1_ref[...])
    hn = h_ref[...] + _dotT(f, w2_ref[...]) + b2_ref[...]
    r = _ln(hn, g_ref[...], b_ref[...])
    o_ref[...] = _dotT(r, wout_ref[...]) + bout_ref[...]


def _vec_spec(n):
    return pl.BlockSpec((n,), lambda i: (0,))


_ROW_SPEC = pl.BlockSpec((BN, D), lambda i: (i, 0))
_HALF_SPEC = pl.BlockSpec((BN, HD), lambda i: (i, 0))
_D16_SPEC = pl.BlockSpec((BN, 16), lambda i: (i, 0))
_RP_SHAPE = jax.ShapeDtypeStruct((N, HD), jnp.bfloat16)

_tc_h = pl.pallas_call(
    _tc_h_body,
    grid=(GRID,),
    in_specs=[
        _ROW_SPEC,
        pl.BlockSpec((D, D), lambda i: (0, 0)),
        _vec_spec(D),
    ],
    out_specs=_ROW_SPEC,
    out_shape=jax.ShapeDtypeStruct((N, D), jnp.float32),
)

_tc_rp = pl.pallas_call(
    _tc_rp_body,
    grid=(GRID,),
    in_specs=[
        _ROW_SPEC,
        _D16_SPEC,
        _D16_SPEC,
        _vec_spec(D),
        _vec_spec(D),
    ],
    out_specs=[_HALF_SPEC, _HALF_SPEC],
    out_shape=[_RP_SHAPE, _RP_SHAPE],
)

_tc_mid = pl.pallas_call(
    _tc_mid_body,
    grid=(GRID,),
    in_specs=[
        _ROW_SPEC,
        _HALF_SPEC,
        _HALF_SPEC,
        _D16_SPEC,
        _D16_SPEC,
        pl.BlockSpec((H, D), lambda i: (0, 0)),
        _vec_spec(H),
        pl.BlockSpec((D, H), lambda i: (0, 0)),
        _vec_spec(D),
        _vec_spec(D),
        _vec_spec(D),
    ],
    out_specs=[_ROW_SPEC, _HALF_SPEC, _HALF_SPEC],
    out_shape=[
        jax.ShapeDtypeStruct((N, D), jnp.float32),
        _RP_SHAPE,
        _RP_SHAPE,
    ],
)

_tc_out = pl.pallas_call(
    _tc_out_body,
    grid=(GRID,),
    in_specs=[
        _ROW_SPEC,
        _HALF_SPEC,
        _HALF_SPEC,
        _D16_SPEC,
        _D16_SPEC,
        pl.BlockSpec((H, D), lambda i: (0, 0)),
        _vec_spec(H),
        pl.BlockSpec((D, H), lambda i: (0, 0)),
        _vec_spec(D),
        _vec_spec(D),
        _vec_spec(D),
        pl.BlockSpec((D, D), lambda i: (0, 0)),
        _vec_spec(D),
    ],
    out_specs=_ROW_SPEC,
    out_shape=jax.ShapeDtypeStruct((N, D), jnp.float32),
)


def kernel(x, edge_index, Win, b_in, ln_g, ln_b, W1, b1, W2, b2,
           out_g, out_b, Wout, b_out):
    seg_sum, deg_count = _sc_kernels()
    src = edge_index[0]
    dst = edge_index[1]
    pad = EPAD - E
    sidx = jnp.concatenate(
        [src, jnp.zeros((pad,), jnp.int32)]).reshape(SCHUNKS, CHS)
    didx = jnp.concatenate(
        [dst, jnp.full((pad,), GARBAGE, jnp.int32)]).reshape(SCHUNKS, CHS)
    degidx = jnp.concatenate(
        [src, jnp.full((pad,), GARBAGE, jnp.int32)]).reshape(TCHUNKS, CH)
    zeros_hd = jnp.zeros((ACC_ROWS, HD), jnp.bfloat16)
    zeros_16 = jnp.zeros((ACC_ROWS, 16), jnp.float32)
    ones_16 = jnp.ones((CH, 16), jnp.float32)

    dlo, dhi = deg_count(degidx, zeros_16, ones_16)
    h = _tc_h(x, Win, b_in)
    rlo, rhi = _tc_rp(h, dlo, dhi, ln_g[0], ln_b[0])
    W1b = W1.astype(jnp.bfloat16)
    out = None
    for l in range(L):
        alo, ahi = seg_sum(rlo, rhi, sidx, didx, zeros_hd)
        if l < L - 1:
            h, rlo, rhi = _tc_mid(h, alo, ahi, dlo, dhi, W1b[l], b1[l],
                                  W2[l], b2[l], ln_g[l + 1], ln_b[l + 1])
        else:
            out = _tc_out(h, alo, ahi, dlo, dhi, W1b[l], b1[l], W2[l], b2[l],
                          out_g, out_b, Wout, b_out)
    return out
